# baseline (speedup 1.0000x reference)
import jax
import jax.numpy as jnp
from jax import lax
from jax.experimental import pallas as pl
from jax.experimental.pallas import tpu as pltpu
from jax.experimental.pallas import tpu_sc as plsc

B, L, F = 4096, 20, 7
O_LIN, O_EMB, O = 16, 8, 24
NC, NS, LANES = 2, 16, 16
NW = NC * NS
CK = 512
NCH = B // CK
NSUB = L * NCH
PER_TILE = NSUB // NW
GB = 8
NGB = CK // (LANES * GB)


def _sc_body(x_hbm, m_hbm, wb_hbm, br_hbm, tab_hbm, out_hbm,
             x_v, m_v, wb_v, br_v, tab_v, o_v0, o_v1, sem0, sem1):
    wid = lax.axis_index("s") * NC + lax.axis_index("c")
    pltpu.sync_copy(wb_hbm, wb_v)
    pltpu.sync_copy(br_hbm, br_v)
    pltpu.sync_copy(tab_hbm, tab_v)

    def compute_subtask(o_v, l, c):
        pltpu.sync_copy(x_hbm.at[l, :, pl.ds(c * CK, CK)], x_v)
        pltpu.sync_copy(m_hbm.at[l, pl.ds(c * CK, CK)], m_v)

        def inner(gb, carry2):
            base = gb * (LANES * GB)
            xm, mv, mapped = [], [], []
            for g in range(GB):
                off = base + g * LANES
                m = m_v[pl.ds(off, LANES)]
                feats = [x_v[f, pl.ds(off, LANES)] for f in range(6)]
                xm.append([feats[f] * m for f in range(5)])
                z = (feats[5] * m).astype(jnp.int32)
                mapped.append(jnp.where((z >= 57) & (z <= 80), z - 56, 0))
                mv.append(m)
            for o in range(O_LIN):
                bo = br_v[pl.ds(o * LANES, LANES)]
                w = [wb_v[pl.ds((o * 5 + f) * LANES, LANES)] for f in range(5)]
                for g in range(GB):
                    acc = bo
                    for f in range(5):
                        acc = acc + xm[g][f] * w[f]
                    o_v[o, pl.ds(base + g * LANES, LANES)] = jnp.maximum(acc, 0.0)
            for g in range(GB):
                m8 = mapped[g] * O_EMB
                for j in range(O_EMB):
                    e = plsc.load_gather(tab_v, [m8 + j])
                    o_v[O_LIN + j, pl.ds(base + g * LANES, LANES)] = e * mv[g]
            return carry2

        lax.fori_loop(0, NGB, inner, 0)

    def subtask(k, carry):
        t = wid * PER_TILE + k
        l = t // NCH
        c = t % NCH

        def run(o_v, sem):
            @pl.when(k >= 2)
            def _wait_prev():
                t2 = t - 2
                pltpu.make_async_copy(
                    o_v, out_hbm.at[t2 // NCH, :, pl.ds((t2 % NCH) * CK, CK)],
                    sem).wait()
            compute_subtask(o_v, l, c)
            pltpu.async_copy(o_v, out_hbm.at[l, :, pl.ds(c * CK, CK)], sem)

        @pl.when(lax.rem(k, 2) == 0)
        def _even():
            run(o_v0, sem0)

        @pl.when(lax.rem(k, 2) == 1)
        def _odd():
            run(o_v1, sem1)

        return carry

    lax.fori_loop(0, PER_TILE, subtask, 0)
    t4 = wid * PER_TILE + (PER_TILE - 1)
    t3 = wid * PER_TILE + (PER_TILE - 2)
    pltpu.make_async_copy(
        o_v0, out_hbm.at[t4 // NCH, :, pl.ds((t4 % NCH) * CK, CK)], sem0).wait()
    pltpu.make_async_copy(
        o_v1, out_hbm.at[t3 // NCH, :, pl.ds((t3 % NCH) * CK, CK)], sem1).wait()


def kernel(elements_info, elements_mask, W, b, tm_table):
    x_t = jnp.transpose(elements_info, (1, 2, 0))
    m_t = jnp.transpose(elements_mask, (1, 0))
    wb = jnp.repeat(W.reshape(-1), LANES)
    br = jnp.repeat(b, LANES)
    tab = jnp.pad(tm_table.reshape(-1), (0, 56))
    mesh = plsc.VectorSubcoreMesh(core_axis_name="c", subcore_axis_name="s")
    out = pl.kernel(
        _sc_body,
        out_type=jax.ShapeDtypeStruct((L, O, B), jnp.float32),
        mesh=mesh,
        compiler_params=pltpu.CompilerParams(needs_layout_passes=False),
        scratch_types=[
            pltpu.VMEM((F, CK), jnp.float32),
            pltpu.VMEM((CK,), jnp.float32),
            pltpu.VMEM((80 * LANES,), jnp.float32),
            pltpu.VMEM((O_LIN * LANES,), jnp.float32),
            pltpu.VMEM((256,), jnp.float32),
            pltpu.VMEM((O, CK), jnp.float32),
            pltpu.VMEM((O, CK), jnp.float32),
            pltpu.SemaphoreType.DMA,
            pltpu.SemaphoreType.DMA,
        ],
    )(x_t, m_t, wb, br, tab)
    return jnp.transpose(out, (2, 0, 1))


# speedup vs baseline: 1.0277x; 1.0277x over previous
import jax
import jax.numpy as jnp
from jax import lax
from jax.experimental import pallas as pl
from jax.experimental.pallas import tpu as pltpu
from jax.experimental.pallas import tpu_sc as plsc

B, L, F = 4096, 20, 7
O_LIN, O_EMB, O = 16, 8, 24
NC, NS, LANES = 2, 16, 16
NW = NC * NS
CK = 512
NCH = B // CK
NSUB = L * NCH
PER_TILE = NSUB // NW
GB = 8
STEP = LANES * GB


def _sc_body(x_hbm, m_hbm, wb_hbm, br_hbm, tab_hbm, out_hbm,
             x_v, m_v, wb_v, br_v, tab_v, o_v):
    wid = lax.axis_index("s") * NC + lax.axis_index("c")
    pltpu.sync_copy(wb_hbm, wb_v)
    pltpu.sync_copy(br_hbm, br_v)
    pltpu.sync_copy(tab_hbm, tab_v)

    def subtask(k, carry):
        t = wid * PER_TILE + k
        l = t // NCH
        c = t % NCH
        pltpu.sync_copy(x_hbm.at[l, :, pl.ds(c * CK, CK)], x_v)
        pltpu.sync_copy(m_hbm.at[l, pl.ds(c * CK, CK)], m_v)

        @plsc.parallel_loop(0, CK, step=STEP, carry=jnp.int32(0))
        def inner(base, carry2):
            xm, mv, mapped = [], [], []
            for g in range(GB):
                off = base + g * LANES
                m = m_v[pl.ds(off, LANES)]
                feats = [x_v[f, pl.ds(off, LANES)] for f in range(6)]
                xm.append([feats[f] * m for f in range(5)])
                z = (feats[5] * m).astype(jnp.int32)
                mapped.append(jnp.where((z >= 57) & (z <= 80), z - 56, 0))
                mv.append(m)
            for o in range(O_LIN):
                bo = br_v[pl.ds(o * LANES, LANES)]
                w = [wb_v[pl.ds((o * 5 + f) * LANES, LANES)] for f in range(5)]
                for g in range(GB):
                    acc = bo
                    for f in range(5):
                        acc = acc + xm[g][f] * w[f]
                    o_v[o, pl.ds(base + g * LANES, LANES)] = jnp.maximum(acc, 0.0)
            for g in range(GB):
                m8 = mapped[g] * O_EMB
                for j in range(O_EMB):
                    e = plsc.load_gather(tab_v, [m8 + j])
                    o_v[O_LIN + j, pl.ds(base + g * LANES, LANES)] = e * mv[g]
            return carry2

        pltpu.sync_copy(o_v, out_hbm.at[l, :, pl.ds(c * CK, CK)])
        return carry

    lax.fori_loop(0, PER_TILE, subtask, 0)


def kernel(elements_info, elements_mask, W, b, tm_table):
    x_t = jnp.transpose(elements_info, (1, 2, 0))
    m_t = jnp.transpose(elements_mask, (1, 0))
    wb = jnp.repeat(W.reshape(-1), LANES)
    br = jnp.repeat(b, LANES)
    tab = jnp.pad(tm_table.reshape(-1), (0, 56))
    mesh = plsc.VectorSubcoreMesh(core_axis_name="c", subcore_axis_name="s")
    out = pl.kernel(
        _sc_body,
        out_type=jax.ShapeDtypeStruct((L, O, B), jnp.float32),
        mesh=mesh,
        compiler_params=pltpu.CompilerParams(needs_layout_passes=False),
        scratch_types=[
            pltpu.VMEM((F, CK), jnp.float32),
            pltpu.VMEM((CK,), jnp.float32),
            pltpu.VMEM((80 * LANES,), jnp.float32),
            pltpu.VMEM((O_LIN * LANES,), jnp.float32),
            pltpu.VMEM((256,), jnp.float32),
            pltpu.VMEM((O, CK), jnp.float32),
        ],
    )(x_t, m_t, wb, br, tab)
    return jnp.transpose(out, (2, 0, 1))


# speedup vs baseline: 1.0693x; 1.0405x over previous
import jax
import jax.numpy as jnp
from jax import lax
from jax.experimental import pallas as pl
from jax.experimental.pallas import tpu as pltpu
from jax.experimental.pallas import tpu_sc as plsc

B, L, F = 4096, 20, 7
O_LIN, O_EMB, O = 16, 8, 24
NC, NS, LANES = 2, 16, 16
WIDE = 2 * LANES
NW = NC * NS
CK = 512
NCH = B // CK
NSUB = L * NCH
PER_TILE = NSUB // NW
GB = 4
STEP = WIDE * GB

_ILV = plsc.PackFormat.INTERLEAVED


def _sc_body(x_hbm, m_hbm, wb_hbm, br_hbm, tab_hbm, out_hbm,
             x_v, m_v, wb_v, br_v, tab_v, o_v):
    wid = lax.axis_index("s") * NC + lax.axis_index("c")
    pltpu.sync_copy(wb_hbm, wb_v)
    pltpu.sync_copy(br_hbm, br_v)
    pltpu.sync_copy(tab_hbm, tab_v)

    def subtask(k, carry):
        t = wid * PER_TILE + k
        l = t // NCH
        c = t % NCH
        pltpu.sync_copy(x_hbm.at[l, :, pl.ds(c * CK, CK)], x_v)
        pltpu.sync_copy(m_hbm.at[l, pl.ds(c * CK, CK)], m_v)

        @plsc.parallel_loop(0, CK, step=STEP, carry=jnp.int32(0))
        def inner(base, carry2):
            xm, m8s, mfs = [], [], []
            for g in range(GB):
                off = base + g * WIDE
                mlo = m_v[pl.ds(off, LANES)]
                mhi = m_v[pl.ds(off + LANES, LANES)]
                mbf = plsc.pack(mlo, mhi, format=_ILV)
                xm.append([
                    plsc.pack(x_v[f, pl.ds(off, LANES)],
                              x_v[f, pl.ds(off + LANES, LANES)],
                              format=_ILV) * mbf
                    for f in range(5)
                ])
                m8s.append([])
                mfs.append([mlo, mhi])
                for h, mf in ((0, mlo), (1, mhi)):
                    z = x_v[5, pl.ds(off + h * LANES, LANES)] * mf
                    zi = z.astype(jnp.int32)
                    mapped = jnp.where((zi >= 57) & (zi <= 80), zi - 56, 0)
                    m8s[g].append(mapped * O_EMB)
            for o in range(O_LIN):
                bo = plsc.bitcast(br_v[pl.ds(o * LANES, LANES)], jnp.bfloat16)
                w = [
                    plsc.bitcast(
                        wb_v[pl.ds((o * 5 + f) * LANES, LANES)], jnp.bfloat16)
                    for f in range(5)
                ]
                for g in range(GB):
                    acc = bo
                    for f in range(5):
                        acc = acc + xm[g][f] * w[f]
                    acc = jnp.maximum(acc, jnp.bfloat16(0))
                    alo, ahi = plsc.unpack(acc, format=_ILV)
                    o_v[o, pl.ds(base + g * WIDE, LANES)] = alo
                    o_v[o, pl.ds(base + g * WIDE + LANES, LANES)] = ahi
            for g in range(GB):
                for h in range(2):
                    m8 = m8s[g][h]
                    mf = mfs[g][h]
                    off = base + g * WIDE + h * LANES
                    for j in range(O_EMB):
                        e = plsc.load_gather(tab_v, [m8 + j])
                        o_v[O_LIN + j, pl.ds(off, LANES)] = e * mf
            return carry2

        pltpu.sync_copy(o_v, out_hbm.at[l, :, pl.ds(c * CK, CK)])
        return carry

    lax.fori_loop(0, PER_TILE, subtask, 0)


def kernel(elements_info, elements_mask, W, b, tm_table):
    x_t = jnp.transpose(elements_info, (1, 2, 0))
    m_t = jnp.transpose(elements_mask, (1, 0))
    def _dup_bf16_words(v):
        bits = jax.lax.bitcast_convert_type(
            v.astype(jnp.bfloat16), jnp.uint16).astype(jnp.uint32)
        return jnp.repeat(bits | (bits << 16), LANES)

    wb = _dup_bf16_words(W.reshape(-1))
    br = _dup_bf16_words(b)
    tab = jnp.pad(tm_table.reshape(-1), (0, 56))
    mesh = plsc.VectorSubcoreMesh(core_axis_name="c", subcore_axis_name="s")
    out = pl.kernel(
        _sc_body,
        out_type=jax.ShapeDtypeStruct((L, O, B), jnp.float32),
        mesh=mesh,
        compiler_params=pltpu.CompilerParams(needs_layout_passes=False),
        scratch_types=[
            pltpu.VMEM((F, CK), jnp.float32),
            pltpu.VMEM((CK,), jnp.float32),
            pltpu.VMEM((80 * LANES,), jnp.uint32),
            pltpu.VMEM((O_LIN * LANES,), jnp.uint32),
            pltpu.VMEM((256,), jnp.float32),
            pltpu.VMEM((O, CK), jnp.float32),
        ],
    )(x_t, m_t, wb, br, tab)
    return jnp.transpose(out, (2, 0, 1))


# speedup vs baseline: 1.2339x; 1.1539x over previous
import jax
import jax.numpy as jnp
from jax import lax
from jax.experimental import pallas as pl
from jax.experimental.pallas import tpu as pltpu
from jax.experimental.pallas import tpu_sc as plsc

B, L, F = 4096, 20, 7
O_LIN, O_EMB, O = 16, 8, 24
NC, NS, LANES = 2, 16, 16
WIDE = 2 * LANES
NW = NC * NS
CK = 512
NCH = B // CK
NSUB = L * NCH
PER_TILE = NSUB // NW
GB = 4
STEP = WIDE * GB

_ILV = plsc.PackFormat.INTERLEAVED


def _sc_body(x_hbm, m_hbm, wb_hbm, br_hbm, tab_hbm, out_hbm,
             x_vs, m_vs, wb_v, br_v, tab_v, o_vs, in_sems, out_sems):
    wid = lax.axis_index("s") * NC + lax.axis_index("c")
    csem = in_sems[0]
    pltpu.async_copy(wb_hbm, wb_v, csem)
    pltpu.async_copy(br_hbm, br_v, csem)
    pltpu.async_copy(tab_hbm, tab_v, csem)
    pltpu.make_async_copy(wb_hbm, wb_v, csem).wait()
    pltpu.make_async_copy(br_hbm, br_v, csem).wait()
    pltpu.make_async_copy(tab_hbm, tab_v, csem).wait()

    def lc(k):
        t = wid * PER_TILE + k
        return t // NCH, t % NCH

    def in_descs(p, k):
        l, c = lc(k)
        return (
            pltpu.make_async_copy(
                x_hbm.at[l, :, pl.ds(c * CK, CK)], x_vs[p], in_sems[p]),
            pltpu.make_async_copy(
                m_hbm.at[l, pl.ds(c * CK, CK)], m_vs[p], in_sems[p]),
        )

    def out_desc(p, k):
        l, c = lc(k)
        return pltpu.make_async_copy(
            o_vs[p], out_hbm.at[l, :, pl.ds(c * CK, CK)], out_sems[p])

    def issue_in(p, k):
        for d in in_descs(p, k):
            d.start()

    issue_in(0, 0)

    def compute_subtask(x_v, m_v, o_v):
        @plsc.parallel_loop(0, CK, step=STEP, carry=jnp.int32(0))
        def inner(base, carry2):
            xm, m8s, mfs = [], [], []
            for g in range(GB):
                off = base + g * WIDE
                mlo = m_v[pl.ds(off, LANES)]
                mhi = m_v[pl.ds(off + LANES, LANES)]
                mbf = plsc.pack(mlo, mhi, format=_ILV)
                xm.append([
                    plsc.pack(x_v[f, pl.ds(off, LANES)],
                              x_v[f, pl.ds(off + LANES, LANES)],
                              format=_ILV) * mbf
                    for f in range(5)
                ])
                m8s.append([])
                mfs.append([mlo, mhi])
                for h, mf in ((0, mlo), (1, mhi)):
                    z = x_v[5, pl.ds(off + h * LANES, LANES)] * mf
                    zi = z.astype(jnp.int32)
                    mapped = jnp.where((zi >= 57) & (zi <= 80), zi - 56, 0)
                    m8s[g].append(mapped * O_EMB)
            for o in range(O_LIN):
                bo = plsc.bitcast(br_v[pl.ds(o * LANES, LANES)], jnp.bfloat16)
                w = [
                    plsc.bitcast(
                        wb_v[pl.ds((o * 5 + f) * LANES, LANES)], jnp.bfloat16)
                    for f in range(5)
                ]
                for g in range(GB):
                    acc = bo
                    for f in range(5):
                        acc = acc + xm[g][f] * w[f]
                    acc = jnp.maximum(acc, jnp.bfloat16(0))
                    alo, ahi = plsc.unpack(acc, format=_ILV)
                    o_v[o, pl.ds(base + g * WIDE, LANES)] = alo
                    o_v[o, pl.ds(base + g * WIDE + LANES, LANES)] = ahi
            for g in range(GB):
                for h in range(2):
                    m8 = m8s[g][h]
                    mf = mfs[g][h]
                    off = base + g * WIDE + h * LANES
                    for j in range(O_EMB):
                        e = plsc.load_gather(tab_v, [m8 + j])
                        o_v[O_LIN + j, pl.ds(off, LANES)] = e * mf
            return carry2

    def subtask(k, carry):
        def run(p):
            for d in in_descs(p, k):
                d.wait()

            @pl.when(k + 1 < PER_TILE)
            def _prefetch():
                issue_in(1 - p, k + 1)

            @pl.when(k >= 2)
            def _drain_prev_out():
                out_desc(p, k - 2).wait()

            compute_subtask(x_vs[p], m_vs[p], o_vs[p])
            out_desc(p, k).start()

        @pl.when(lax.rem(k, 2) == 0)
        def _even():
            run(0)

        @pl.when(lax.rem(k, 2) == 1)
        def _odd():
            run(1)

        return carry

    lax.fori_loop(0, PER_TILE, subtask, 0)
    out_desc((PER_TILE - 1) % 2, PER_TILE - 1).wait()
    out_desc((PER_TILE - 2) % 2, PER_TILE - 2).wait()


def kernel(elements_info, elements_mask, W, b, tm_table):
    x_t = jnp.transpose(elements_info, (1, 2, 0))
    m_t = jnp.transpose(elements_mask, (1, 0))

    def _dup_bf16_words(v):
        bits = jax.lax.bitcast_convert_type(
            v.astype(jnp.bfloat16), jnp.uint16).astype(jnp.uint32)
        return jnp.repeat(bits | (bits << 16), LANES)

    wb = _dup_bf16_words(W.reshape(-1))
    br = _dup_bf16_words(b)
    tab = jnp.pad(tm_table.reshape(-1), (0, 56))
    mesh = plsc.VectorSubcoreMesh(core_axis_name="c", subcore_axis_name="s")
    out = pl.kernel(
        _sc_body,
        out_type=jax.ShapeDtypeStruct((L, O, B), jnp.float32),
        mesh=mesh,
        compiler_params=pltpu.CompilerParams(needs_layout_passes=False),
        scratch_types=[
            [pltpu.VMEM((F, CK), jnp.float32) for _ in range(2)],
            [pltpu.VMEM((CK,), jnp.float32) for _ in range(2)],
            pltpu.VMEM((80 * LANES,), jnp.uint32),
            pltpu.VMEM((O_LIN * LANES,), jnp.uint32),
            pltpu.VMEM((256,), jnp.float32),
            [pltpu.VMEM((O, CK), jnp.float32) for _ in range(2)],
            [pltpu.SemaphoreType.DMA for _ in range(2)],
            [pltpu.SemaphoreType.DMA for _ in range(2)],
        ],
    )(x_t, m_t, wb, br, tab)
    return jnp.transpose(out, (2, 0, 1))
